# Initial kernel scaffold; baseline (speedup 1.0000x reference)
import jax
import jax.numpy as jnp
from jax.experimental import pallas as pl


def kernel(image, patch_W, patch_b, cls_token, pos_embed, ln1_s, ln1_b, qkv_W, qkv_b, proj_W, proj_b, ln2_s, ln2_b, mlp_W1, mlp_b1, mlp_W2, mlp_b2, norm_s, norm_b, head_W, head_b):
    raise NotImplementedError("write your pallas kernel here")


# speedup vs baseline: 1.3539x; 1.3539x over previous
import functools

import jax
import jax.numpy as jnp
from jax import lax
from jax.experimental import pallas as pl
from jax.experimental.pallas import tpu as pltpu
from jax.experimental.pallas import tpu_sc as plsc

D = 768
L = 12
H = 12
DH = 64
P = 16
G = 14
NPATCH = G * G
NTOK = NPATCH + 1
PRUNE_LAYER = 6
N_KEEP = NPATCH // 2
BATCH = 8
N1 = 208
N2 = 104
NREAL2 = N_KEEP + 1
HID = 4 * D
HCHUNK = 1024
SCALE = 1.0 / (DH ** 0.5)
NEG = -1e30

f32 = jnp.float32
bf16 = jnp.bfloat16
i32 = jnp.int32


_NT = (((1,), (1,)), ((), ()))


def _mm3(a, bh):
    return jnp.dot(a.astype(bf16), bh, preferred_element_type=f32)


def _mm3_nt(a, b):
    return lax.dot_general(a.astype(bf16), b.astype(bf16),
                           dimension_numbers=_NT,
                           preferred_element_type=f32)


def _mm3_vals(a, b):
    return jnp.dot(a.astype(bf16), b.astype(bf16),
                   preferred_element_type=f32)


def _ln(x, s, b):
    mu = jnp.mean(x, axis=-1, keepdims=True)
    var = jnp.mean((x - mu) ** 2, axis=-1, keepdims=True)
    return (x - mu) / jnp.sqrt(var + 1e-6) * s + b


def _attn_body(npad, nreal, embed, *refs):
    if embed:
        (xu_ref, pwh_ref, posc_ref, ln1s_ref, ln1b_ref,
         qwh_ref, qb_ref, owh_ref, ob_ref,
         out_ref, qkv_ref) = refs
    else:
        (x_ref, ln1s_ref, ln1b_ref,
         qwh_ref, qb_ref, owh_ref, ob_ref,
         out_ref, qkv_ref) = refs

    kmask = jnp.where(
        lax.broadcasted_iota(i32, (1, npad), 1) < nreal, 0.0, NEG)

    def per_sample(i, _):
        if embed:
            x_s = _mm3(xu_ref[i], pwh_ref[...]) + posc_ref[...]
        else:
            x_s = x_ref[i]
        h = _ln(x_s, ln1s_ref[...], ln1b_ref[...])
        qkv_ref[...] = _mm3(h, qwh_ref[...]) + qb_ref[...]
        outs = []
        for hd in range(H):
            c = hd * DH
            q = qkv_ref[:, c:c + DH]
            k = qkv_ref[:, D + c:D + c + DH]
            v = qkv_ref[:, 2 * D + c:2 * D + c + DH]
            lg = _mm3_nt(q, k) * SCALE + kmask
            m = jnp.max(lg, axis=-1, keepdims=True)
            e = jnp.exp(lg - m)
            a = e / jnp.sum(e, axis=-1, keepdims=True)
            outs.append(_mm3_vals(a, v))
        o_s = jnp.concatenate(outs, axis=-1)
        out_ref[i] = x_s + _mm3(o_s, owh_ref[...]) + ob_ref[...]
        return 0

    lax.fori_loop(0, BATCH, per_sample, 0)


def _attn_call(npad, nreal, x_or_xu, embed_args, lp):
    (ln1s, ln1b, qwh, qb, owh, ob) = lp
    embed = embed_args is not None
    body = functools.partial(_attn_body, npad, nreal, embed)
    full3 = lambda a: pl.BlockSpec(a.shape, lambda g: (0,) * a.ndim)
    if embed:
        pwh, posc = embed_args
        ins = (x_or_xu, pwh, posc, ln1s, ln1b, qwh, qb, owh, ob)
    else:
        ins = (x_or_xu, ln1s, ln1b, qwh, qb, owh, ob)
    return pl.pallas_call(
        body,
        grid=(1,),
        in_specs=[full3(a) for a in ins],
        out_specs=pl.BlockSpec((BATCH, npad, D), lambda g: (0, 0, 0)),
        out_shape=jax.ShapeDtypeStruct((BATCH, npad, D), f32),
        scratch_shapes=[pltpu.VMEM((npad, 3 * D), f32)],
    )(*ins)


def _mlp_body(npad, variant, *refs):
    if variant == "head":
        (x_ref, ln2s_ref, ln2b_ref, w1h_ref, b1_ref,
         w2h_ref, b2_ref, ns_ref, nb_ref, hwh_ref, hb_ref,
         out_ref, log_ref, acc_ref) = refs
    elif variant == "norms":
        (x_ref, ln2s_ref, ln2b_ref, w1h_ref, b1_ref,
         w2h_ref, b2_ref, out_ref, nrm_ref, acc_ref) = refs
    else:
        (x_ref, ln2s_ref, ln2b_ref, w1h_ref, b1_ref,
         w2h_ref, b2_ref, out_ref, acc_ref) = refs
    j = pl.program_id(0)
    nj = pl.num_programs(0)

    def per_sample(i, _):
        x_s = x_ref[i]
        h2 = _ln(x_s, ln2s_ref[...], ln2b_ref[...])
        hid = _mm3(h2, w1h_ref[...]) + b1_ref[...]
        hid = hid * 0.5 * (1.0 + lax.erf(hid * (2.0 ** -0.5)))
        contrib = _mm3(hid, w2h_ref[...])

        @pl.when(j == 0)
        def _():
            acc_ref[i] = contrib

        @pl.when(j > 0)
        def _():
            acc_ref[i] = acc_ref[i] + contrib
        return 0

    lax.fori_loop(0, BATCH, per_sample, 0)

    @pl.when(j == nj - 1)
    def _():
        def finish(i, _):
            xo = x_ref[i] + acc_ref[i] + b2_ref[...]
            out_ref[i] = xo
            if variant == "norms":
                a = jnp.sum(jnp.abs(xo), axis=-1, keepdims=True)
                rid = lax.broadcasted_iota(i32, (npad, 1), 0)
                valid = (rid >= 1) & (rid <= NPATCH)
                a = jnp.where(valid, a, -1.0)
                nrm_ref[pl.ds(i, 1), :] = jnp.transpose(a)
            return 0

        lax.fori_loop(0, BATCH, finish, 0)

        if variant == "head":
            cls = jnp.concatenate([out_ref[s, 0:1, :] for s in range(BATCH)],
                                  axis=0)
            hcls = _ln(cls, ns_ref[...], nb_ref[...])
            log_ref[...] = _mm3(hcls, hwh_ref[...]) + hb_ref[...]


def _mlp_call(npad, variant, x, lp, extra=()):
    (ln2s, ln2b, w1h, b1, w2h, b2) = lp
    body = functools.partial(_mlp_body, npad, variant)
    njc = HID // HCHUNK
    const = lambda a: pl.BlockSpec(a.shape, lambda j: (0,) * a.ndim)
    ins = [x, ln2s, ln2b, w1h, b1, w2h, b2]
    in_specs = [const(x), const(ln2s), const(ln2b),
                pl.BlockSpec((D, HCHUNK), lambda j: (0, j)),
                pl.BlockSpec((1, HCHUNK), lambda j: (0, j)),
                pl.BlockSpec((HCHUNK, D), lambda j: (j, 0)),
                const(b2)]
    out_shape = [jax.ShapeDtypeStruct((BATCH, npad, D), f32)]
    out_specs = [pl.BlockSpec((BATCH, npad, D), lambda j: (0, 0, 0))]
    if variant == "norms":
        out_shape.append(jax.ShapeDtypeStruct((BATCH, N1), f32))
        out_specs.append(pl.BlockSpec((BATCH, N1), lambda j: (0, 0)))
    if variant == "head":
        ins += list(extra)
        in_specs += [const(a) for a in extra]
        out_shape.append(jax.ShapeDtypeStruct((BATCH, 128), f32))
        out_specs.append(pl.BlockSpec((BATCH, 128), lambda j: (0, 0)))
    res = pl.pallas_call(
        body,
        grid=(njc,),
        in_specs=in_specs,
        out_specs=out_specs,
        out_shape=out_shape,
        scratch_shapes=[pltpu.VMEM((BATCH, npad, D), f32)],
        compiler_params=pltpu.CompilerParams(
            dimension_semantics=("arbitrary",)),
    )(*ins)
    return res[0] if len(res) == 1 else tuple(res)


_GDN = lax.GatherDimensionNumbers(
    offset_dims=(), collapsed_slice_dims=(0,), start_index_map=(0,))


def _shift_up(c, sh, lanes):
    idx = jnp.maximum(lanes - sh, 0)
    g = lax.gather(c, idx[:, None], _GDN, (1,),
                   mode=lax.GatherScatterMode.PROMISE_IN_BOUNDS)
    return jnp.where(lanes >= sh, g, 0)


def _prefix(m, lanes):
    c = m.astype(i32)
    for sh in (1, 2, 4, 8):
        c = c + _shift_up(c, sh, lanes)
    return c


def _sc_prune(keys, xflat):
    mesh = plsc.VectorSubcoreMesh(core_axis_name="c", subcore_axis_name="s")
    nchunks = N1 // 16

    @functools.partial(
        pl.kernel,
        out_type=jax.ShapeDtypeStruct((BATCH, N2, D), f32),
        mesh=mesh,
        scratch_types=[
            pltpu.VMEM((N1,), i32),
            pltpu.VMEM((N2,), i32),
            pltpu.VMEM((N2, D), f32),
            pltpu.SemaphoreType.DMA,
        ],
        compiler_params=pltpu.CompilerParams(needs_layout_passes=False),
    )
    def k(keys_hbm, x_hbm, out_hbm, keys_v, idx_v, rows_v, sem):
        wid = lax.axis_index("s") * 2 + lax.axis_index("c")

        @pl.when(wid < BATCH)
        def _():
            b = wid
            pltpu.sync_copy(keys_hbm.at[b], keys_v)
            lanes = lax.iota(i32, 16)
            nk = jnp.full((16,), N_KEEP, i32)

            def count_ge(t):
                acc = jnp.zeros((16,), i32)
                for kk in range(nchunks):
                    key = keys_v[pl.ds(kk * 16, 16)]
                    acc = acc + plsc.all_reduce_population_count(key >= t)
                return acc

            def bs_step(_, carry):
                lo, hi = carry
                mid = lo + lax.shift_right_arithmetic(
                    hi - lo + 1, jnp.full((16,), 1, i32))
                big = count_ge(mid) >= nk
                return (jnp.where(big, mid, lo), jnp.where(big, hi, mid - 1))

            lo, hi = lax.fori_loop(
                0, 31, bs_step,
                (jnp.zeros((16,), i32), jnp.full((16,), 0x7F800000, i32)))
            vstar = lo
            need = nk - count_ge(vstar + 1)

            clsrow = jnp.full((16,), b * N1, i32)
            for kk in range(N2 // 16):
                idx_v[pl.ds(kk * 16, 16)] = clsrow
            plsc.store_scatter(idx_v, [96 + lanes], clsrow, mask=lanes < N2 - 96)

            running = jnp.zeros((16,), i32)
            eq_seen = jnp.zeros((16,), i32)
            for kk in range(nchunks):
                key = keys_v[pl.ds(kk * 16, 16)]
                m_gt = key > vstar
                m_eq = key == vstar
                eq_rank = _prefix(m_eq, lanes) + eq_seen
                keep = m_gt | (m_eq & (eq_rank <= need))
                slot = _prefix(keep, lanes) + running
                ids = b * N1 + kk * 16 + lanes
                plsc.store_scatter(idx_v, [slot], ids, mask=keep)
                running = running + plsc.all_reduce_population_count(keep)
                eq_seen = eq_seen + plsc.all_reduce_population_count(m_eq)

            pltpu.async_copy(x_hbm.at[idx_v], rows_v, sem).wait()
            pltpu.sync_copy(rows_v, out_hbm.at[b])

    return k(keys, xflat)


def kernel(image, patch_W, patch_b, cls_token, pos_embed, ln1_s, ln1_b, qkv_W,
           qkv_b, proj_W, proj_b, ln2_s, ln2_b, mlp_W1, mlp_b1, mlp_W2,
           mlp_b2, norm_s, norm_b, head_W, head_b):
    xu = image.reshape(BATCH, 3, G, P, G, P).transpose(0, 2, 4, 1, 3, 5)
    xu = xu.reshape(BATCH, NPATCH, 3 * P * P)
    xu_pad = jnp.pad(xu, ((0, 0), (1, N1 - NTOK), (0, 0)))

    posc = jnp.zeros((N1, D), f32)
    posc = posc.at[0].set(cls_token[0, 0] + pos_embed[0, 0])
    posc = posc.at[1:NTOK].set(pos_embed[0, 1:] + patch_b)

    pwh = patch_W.astype(bf16)
    qwh = qkv_W.astype(bf16)
    owh = proj_W.astype(bf16)
    w1h = mlp_W1.astype(bf16)
    w2h = mlp_W2.astype(bf16)
    hWp = jnp.zeros((D, 128), f32).at[:, :head_W.shape[1]].set(head_W)
    hwh = hWp.astype(bf16)
    hbp = jnp.zeros((1, 128), f32).at[0, :head_b.shape[0]].set(head_b)

    r2 = lambda a: a.reshape(1, -1)

    def layer_params_attn(i):
        return (r2(ln1_s[i]), r2(ln1_b[i]), qwh[i], r2(qkv_b[i]),
                owh[i], r2(proj_b[i]))

    def layer_params_mlp(i):
        return (r2(ln2_s[i]), r2(ln2_b[i]), w1h[i], r2(mlp_b1[i]),
                w2h[i], r2(mlp_b2[i]))

    x = _attn_call(N1, NTOK, xu_pad, (pwh, posc), layer_params_attn(0))
    x = _mlp_call(N1, None, x, layer_params_mlp(0))
    for i in range(1, PRUNE_LAYER):
        x = _attn_call(N1, NTOK, x, None, layer_params_attn(i))
        x = _mlp_call(N1, None, x, layer_params_mlp(i))
    x = _attn_call(N1, NTOK, x, None, layer_params_attn(PRUNE_LAYER))
    x, norms = _mlp_call(N1, "norms", x, layer_params_mlp(PRUNE_LAYER))

    keys = lax.bitcast_convert_type(norms, i32)
    x2 = _sc_prune(keys, x.reshape(BATCH * N1, D))

    for i in range(PRUNE_LAYER + 1, L - 1):
        x2 = _attn_call(N2, NREAL2, x2, None, layer_params_attn(i))
        x2 = _mlp_call(N2, None, x2, layer_params_mlp(i))
    x2 = _attn_call(N2, NREAL2, x2, None, layer_params_attn(L - 1))
    _, logits = _mlp_call(N2, "head", x2, layer_params_mlp(L - 1),
                          extra=(r2(norm_s), r2(norm_b), hwh, hbp))
    return logits[:, :head_W.shape[1]]


# speedup vs baseline: 1.4673x; 1.0838x over previous
import functools

import jax
import jax.numpy as jnp
from jax import lax
from jax.experimental import pallas as pl
from jax.experimental.pallas import tpu as pltpu
from jax.experimental.pallas import tpu_sc as plsc

D = 768
L = 12
H = 12
DH = 64
P = 16
G = 14
NPATCH = G * G
NTOK = NPATCH + 1
PRUNE_LAYER = 6
N_KEEP = NPATCH // 2
BATCH = 8
N1 = 208
N2 = 104
NREAL2 = N_KEEP + 1
HID = 4 * D
HCHUNK = 1024
NPH = 4
SCALE = 1.0 / (DH ** 0.5)
NEG = -1e30

f32 = jnp.float32
bf16 = jnp.bfloat16
i32 = jnp.int32

_NT = (((1,), (1,)), ((), ()))


def _mm(a, bh):
    return jnp.dot(a.astype(bf16), bh.astype(bf16), preferred_element_type=f32)


def _mm_nt(a, b):
    return lax.dot_general(a.astype(bf16), b.astype(bf16),
                           dimension_numbers=_NT,
                           preferred_element_type=f32)


def _ln(x, s, b):
    mu = jnp.mean(x, axis=-1, keepdims=True)
    var = jnp.mean((x - mu) ** 2, axis=-1, keepdims=True)
    return (x - mu) / jnp.sqrt(var + 1e-6) * s + b


def _stage_body(npad, nreal, stage1, nlayers, *refs):
    if stage1:
        (xin_ref, pwh_ref, posc_ref,
         ln1s_ref, ln1b_ref, qw_ref, qb_ref, ow_ref, ob_ref,
         ln2s_ref, ln2b_ref, w1_ref, b1_ref, w2_ref, b2_ref,
         out_ref, nrm_ref, xs_ref, qkv_ref, acc_ref) = refs
    else:
        (xin_ref,
         ln1s_ref, ln1b_ref, qw_ref, qb_ref, ow_ref, ob_ref,
         ln2s_ref, ln2b_ref, w1_ref, b1_ref, w2_ref, b2_ref,
         ns_ref, nb_ref, hw_ref, hb_ref,
         out_ref, xs_ref, qkv_ref, acc_ref) = refs

    li = pl.program_id(0)
    p = pl.program_id(1)

    kmask = jnp.where(
        lax.broadcasted_iota(i32, (1, npad), 1) < nreal, 0.0, NEG)

    @pl.when((li == 0) & (p == 0))
    def _():
        if stage1:
            def emb(i, _):
                xs_ref[i] = _mm(xin_ref[i], pwh_ref[...]) + posc_ref[...]
                return 0
            lax.fori_loop(0, BATCH, emb, 0)
        else:
            xs_ref[...] = xin_ref[...]

    @pl.when(p == 0)
    def _():
        def attn(i, _):
            x_s = xs_ref[i]
            h = _ln(x_s, ln1s_ref[0], ln1b_ref[0])
            qkv_ref[...] = _mm(h, qw_ref[0]) + qb_ref[0]
            outs = []
            for hd in range(H):
                c = hd * DH
                q = qkv_ref[:, c:c + DH]
                k = qkv_ref[:, D + c:D + c + DH]
                v = qkv_ref[:, 2 * D + c:2 * D + c + DH]
                lg = _mm_nt(q, k) * SCALE + kmask
                m = jnp.max(lg, axis=-1, keepdims=True)
                e = jnp.exp(lg - m)
                a = e / jnp.sum(e, axis=-1, keepdims=True)
                outs.append(_mm(a, v))
            o_s = jnp.concatenate(outs, axis=-1)
            xs_ref[i] = x_s + _mm(o_s, ow_ref[0]) + ob_ref[0]
            return 0

        lax.fori_loop(0, BATCH, attn, 0)

    @pl.when(p > 0)
    def _():
        def mlp(i, _):
            x_s = xs_ref[i]
            h2 = _ln(x_s, ln2s_ref[0], ln2b_ref[0])
            hid = _mm(h2, w1_ref[0]) + b1_ref[0]
            hid = hid * 0.5 * (1.0 + lax.erf(hid * (2.0 ** -0.5)))
            contrib = _mm(hid, w2_ref[0])

            @pl.when(p == 1)
            def _():
                acc_ref[i] = contrib

            @pl.when(p > 1)
            def _():
                acc_ref[i] = acc_ref[i] + contrib

            @pl.when(p == NPH - 1)
            def _():
                xs_ref[i] = x_s + acc_ref[i] + b2_ref[0]
            return 0

        lax.fori_loop(0, BATCH, mlp, 0)

    @pl.when((li == nlayers - 1) & (p == NPH - 1))
    def _():
        if stage1:
            def finish(i, _):
                xo = xs_ref[i]
                out_ref[i] = xo
                a = jnp.sum(jnp.abs(xo), axis=-1, keepdims=True)
                rid = lax.broadcasted_iota(i32, (npad, 1), 0)
                valid = (rid >= 1) & (rid <= NPATCH)
                a = jnp.where(valid, a, -1.0)
                nrm_ref[pl.ds(i, 1), :] = jnp.transpose(a)
                return 0

            lax.fori_loop(0, BATCH, finish, 0)
        else:
            cls = jnp.concatenate([xs_ref[s, 0:1, :] for s in range(BATCH)],
                                  axis=0)
            hcls = _ln(cls, ns_ref[...], nb_ref[...])
            out_ref[...] = _mm(hcls, hw_ref[...]) + hb_ref[...]


def _stage_call(npad, nreal, stage1, nlayers, xin, wp, extra=()):
    (ln1s, ln1b, qw, qb, ow, ob, ln2s, ln2b, w1, b1, w2, b2) = wp
    body = functools.partial(_stage_body, npad, nreal, stage1, nlayers)
    const = lambda a: pl.BlockSpec(a.shape, lambda l, p: (0,) * a.ndim)
    perl2 = lambda a: pl.BlockSpec((1,) + a.shape[1:], lambda l, p: (l,) + (0,) * (a.ndim - 1))
    c01 = lambda p: jnp.maximum(p - 1, 0)
    ins = [xin]
    in_specs = [const(xin)]
    if stage1:
        pwh, posc = extra[:2]
        ins += [pwh, posc]
        in_specs += [const(pwh), const(posc)]
    ins += [ln1s, ln1b, qw, qb, ow, ob, ln2s, ln2b]
    in_specs += [perl2(ln1s), perl2(ln1b), perl2(qw), perl2(qb),
                 perl2(ow), perl2(ob), perl2(ln2s), perl2(ln2b)]
    ins += [w1, b1, w2, b2]
    in_specs += [
        pl.BlockSpec((1, D, HCHUNK), lambda l, p: (l, 0, c01(p))),
        pl.BlockSpec((1, 1, HCHUNK), lambda l, p: (l, 0, c01(p))),
        pl.BlockSpec((1, HCHUNK, D), lambda l, p: (l, c01(p), 0)),
        perl2(b2),
    ]
    if stage1:
        out_shape = [jax.ShapeDtypeStruct((BATCH, npad, D), f32),
                     jax.ShapeDtypeStruct((BATCH, N1), f32)]
        out_specs = [pl.BlockSpec((BATCH, npad, D), lambda l, p: (0, 0, 0)),
                     pl.BlockSpec((BATCH, N1), lambda l, p: (0, 0))]
    else:
        ns, nb, hw, hb = extra
        ins += [ns, nb, hw, hb]
        in_specs += [const(ns), const(nb), const(hw), const(hb)]
        out_shape = [jax.ShapeDtypeStruct((BATCH, 128), f32)]
        out_specs = [pl.BlockSpec((BATCH, 128), lambda l, p: (0, 0))]
    res = pl.pallas_call(
        body,
        grid=(nlayers, NPH),
        in_specs=in_specs,
        out_specs=out_specs,
        out_shape=out_shape,
        scratch_shapes=[pltpu.VMEM((BATCH, npad, D), f32),
                        pltpu.VMEM((npad, 3 * D), f32),
                        pltpu.VMEM((BATCH, npad, D), f32)],
        compiler_params=pltpu.CompilerParams(
            dimension_semantics=("arbitrary", "arbitrary")),
    )(*ins)
    return res[0] if len(res) == 1 else tuple(res)


_GDN = lax.GatherDimensionNumbers(
    offset_dims=(), collapsed_slice_dims=(0,), start_index_map=(0,))


def _shift_up(c, sh, lanes):
    idx = jnp.maximum(lanes - sh, 0)
    g = lax.gather(c, idx[:, None], _GDN, (1,),
                   mode=lax.GatherScatterMode.PROMISE_IN_BOUNDS)
    return jnp.where(lanes >= sh, g, 0)


def _prefix(m, lanes):
    c = m.astype(i32)
    for sh in (1, 2, 4, 8):
        c = c + _shift_up(c, sh, lanes)
    return c


def _sc_prune(keys, xflat):
    mesh = plsc.VectorSubcoreMesh(core_axis_name="c", subcore_axis_name="s")
    nchunks = N1 // 16

    @functools.partial(
        pl.kernel,
        out_type=jax.ShapeDtypeStruct((BATCH, N2, D), f32),
        mesh=mesh,
        scratch_types=[
            pltpu.VMEM((N1,), i32),
            pltpu.VMEM((N2,), i32),
            pltpu.VMEM((N2, D), f32),
            pltpu.SemaphoreType.DMA,
        ],
        compiler_params=pltpu.CompilerParams(needs_layout_passes=False),
    )
    def k(keys_hbm, x_hbm, out_hbm, keys_v, idx_v, rows_v, sem):
        wid = lax.axis_index("s") * 2 + lax.axis_index("c")

        @pl.when(wid < BATCH)
        def _():
            b = wid
            pltpu.sync_copy(keys_hbm.at[b], keys_v)
            lanes = lax.iota(i32, 16)
            nk = jnp.full((16,), N_KEEP, i32)

            def count_ge(t):
                acc = jnp.zeros((16,), i32)
                for kk in range(nchunks):
                    key = keys_v[pl.ds(kk * 16, 16)]
                    acc = acc + plsc.all_reduce_population_count(key >= t)
                return acc

            def bs_step(_, carry):
                lo, hi = carry
                mid = lo + lax.shift_right_arithmetic(
                    hi - lo + 1, jnp.full((16,), 1, i32))
                big = count_ge(mid) >= nk
                return (jnp.where(big, mid, lo), jnp.where(big, hi, mid - 1))

            lo, hi = lax.fori_loop(
                0, 31, bs_step,
                (jnp.zeros((16,), i32), jnp.full((16,), 0x7F800000, i32)))
            vstar = lo
            need = nk - count_ge(vstar + 1)

            clsrow = jnp.full((16,), b * N1, i32)
            for kk in range(N2 // 16):
                idx_v[pl.ds(kk * 16, 16)] = clsrow
            plsc.store_scatter(idx_v, [96 + lanes], clsrow,
                               mask=lanes < N2 - 96)

            running = jnp.zeros((16,), i32)
            eq_seen = jnp.zeros((16,), i32)
            for kk in range(nchunks):
                key = keys_v[pl.ds(kk * 16, 16)]
                m_gt = key > vstar
                m_eq = key == vstar
                eq_rank = _prefix(m_eq, lanes) + eq_seen
                keep = m_gt | (m_eq & (eq_rank <= need))
                slot = _prefix(keep, lanes) + running
                ids = b * N1 + kk * 16 + lanes
                plsc.store_scatter(idx_v, [slot], ids, mask=keep)
                running = running + plsc.all_reduce_population_count(keep)
                eq_seen = eq_seen + plsc.all_reduce_population_count(m_eq)

            pltpu.async_copy(x_hbm.at[idx_v], rows_v, sem).wait()
            pltpu.sync_copy(rows_v, out_hbm.at[b])

    return k(keys, xflat)


def kernel(image, patch_W, patch_b, cls_token, pos_embed, ln1_s, ln1_b, qkv_W,
           qkv_b, proj_W, proj_b, ln2_s, ln2_b, mlp_W1, mlp_b1, mlp_W2,
           mlp_b2, norm_s, norm_b, head_W, head_b):
    xu = image.reshape(BATCH, 3, G, P, G, P).transpose(0, 2, 4, 1, 3, 5)
    xu = xu.reshape(BATCH, NPATCH, 3 * P * P)
    xu_pad = jnp.pad(xu, ((0, 0), (1, N1 - NTOK), (0, 0))).astype(bf16)

    posc = jnp.zeros((N1, D), f32)
    posc = posc.at[0].set(cls_token[0, 0] + pos_embed[0, 0])
    posc = posc.at[1:NTOK].set(pos_embed[0, 1:] + patch_b)

    pwh = patch_W.astype(bf16)
    qwh = qkv_W.astype(bf16)
    owh = proj_W.astype(bf16)
    w2h = mlp_W2.astype(bf16)
    hWp = jnp.zeros((D, 128), f32).at[:, :head_W.shape[1]].set(head_W)
    hwh = hWp.astype(bf16)
    hbp = jnp.zeros((1, 128), f32).at[0, :head_b.shape[0]].set(head_b)

    r2 = lambda a: a.reshape(1, -1)

    def wp2(lo, hi):
        sl = slice(lo, hi)
        return (ln1_s[sl, None], ln1_b[sl, None], qwh[sl], qkv_b[sl, None],
                owh[sl], proj_b[sl, None], ln2_s[sl, None], ln2_b[sl, None],
                mlp_W1[sl].astype(bf16), mlp_b1[sl, None], w2h[sl],
                mlp_b2[sl, None])

    x, norms = _stage_call(N1, NTOK, True, PRUNE_LAYER + 1, xu_pad,
                           wp2(0, PRUNE_LAYER + 1), extra=(pwh, posc))

    keys = lax.bitcast_convert_type(norms, i32)
    x2 = _sc_prune(keys, x.reshape(BATCH * N1, D))

    logits = _stage_call(N2, NREAL2, False, L - PRUNE_LAYER - 1, x2,
                         wp2(PRUNE_LAYER + 1, L),
                         extra=(r2(norm_s), r2(norm_b), hwh, hbp))
    return logits[:, :head_W.shape[1]]


# speedup vs baseline: 1.5327x; 1.0446x over previous
import functools

import jax
import jax.numpy as jnp
from jax import lax
from jax.experimental import pallas as pl
from jax.experimental.pallas import tpu as pltpu
from jax.experimental.pallas import tpu_sc as plsc

D = 768
L = 12
H = 12
DH = 64
P = 16
G = 14
NPATCH = G * G
NTOK = NPATCH + 1
PRUNE_LAYER = 6
N_KEEP = NPATCH // 2
BATCH = 8
N1 = 208
N2 = 112
NREAL2 = N_KEEP + 1
HID = 4 * D
HCHUNK = 1024
NPH = 4
SCALE = 1.0 / (DH ** 0.5)
NEG = -1e30

f32 = jnp.float32
bf16 = jnp.bfloat16
i32 = jnp.int32

_NT = (((1,), (1,)), ((), ()))


def _mm(a, bh):
    return jnp.dot(a.astype(bf16), bh.astype(bf16), preferred_element_type=f32)


def _mm_nt(a, b):
    return lax.dot_general(a.astype(bf16), b.astype(bf16),
                           dimension_numbers=_NT,
                           preferred_element_type=f32)


def _ln(x, s, b):
    mu = jnp.mean(x, axis=-1, keepdims=True)
    var = jnp.mean((x - mu) ** 2, axis=-1, keepdims=True)
    return (x - mu) / jnp.sqrt(var + 1e-6) * s + b


def _stage_body(npad, nreal, stage1, nlayers, *refs):
    if stage1:
        (xin_ref, pwh_ref, posc_ref,
         ln1s_ref, ln1b_ref, qw_ref, qb_ref, ow_ref, ob_ref,
         ln2s_ref, ln2b_ref, w1_ref, b1_ref, w2_ref, b2_ref,
         out_ref, nrm_ref, xs_ref, h_ref, qkv_ref, hg_ref, mm_ref,
         acc_ref) = refs
    else:
        (xin_ref,
         ln1s_ref, ln1b_ref, qw_ref, qb_ref, ow_ref, ob_ref,
         ln2s_ref, ln2b_ref, w1_ref, b1_ref, w2_ref, b2_ref,
         ns_ref, nb_ref, hw_ref, hb_ref,
         out_ref, xs_ref, h_ref, qkv_ref, hg_ref, mm_ref, acc_ref) = refs

    li = pl.program_id(0)
    p = pl.program_id(1)
    R = BATCH * npad

    kmask = jnp.where(
        lax.broadcasted_iota(i32, (1, npad), 1) < nreal, 0.0, NEG)

    @pl.when((li == 0) & (p == 0))
    def _():
        if stage1:
            def emb(i, _):
                xs_ref[pl.ds(pl.multiple_of(i * npad, 16), npad)] = (
                    _mm(xin_ref[i], pwh_ref[...]) + posc_ref[...])
                return 0
            lax.fori_loop(0, BATCH, emb, 0)
        else:
            def cp(i, _):
                xs_ref[pl.ds(pl.multiple_of(i * npad, 16), npad)] = xin_ref[i]
                return 0
            lax.fori_loop(0, BATCH, cp, 0)

    @pl.when(p == 0)
    def _():
        def ln1(i, _):
            r = pl.ds(pl.multiple_of(i * npad, 16), npad)
            h_ref[r] = _ln(xs_ref[r], ln1s_ref[0], ln1b_ref[0]).astype(bf16)
            return 0

        lax.fori_loop(0, BATCH, ln1, 0)
        qkv_ref[...] = (jnp.dot(h_ref[...], qw_ref[0],
                                preferred_element_type=f32)
                        + qb_ref[0]).astype(bf16)

        def attn(i, _):
            r = pl.ds(pl.multiple_of(i * npad, 16), npad)
            outs = []
            for hd in range(H):
                c = hd * DH
                q = qkv_ref[r, c:c + DH]
                k = qkv_ref[r, D + c:D + c + DH]
                v = qkv_ref[r, 2 * D + c:2 * D + c + DH]
                lg = _mm_nt(q, k) * SCALE + kmask
                m = jnp.max(lg, axis=-1, keepdims=True)
                e = jnp.exp(lg - m)
                a = e / jnp.sum(e, axis=-1, keepdims=True)
                outs.append(_mm(a, v))
            h_ref[r] = jnp.concatenate(outs, axis=-1).astype(bf16)
            return 0

        lax.fori_loop(0, BATCH, attn, 0)
        mm_ref[...] = jnp.dot(h_ref[...], ow_ref[0],
                              preferred_element_type=f32)
        xs_ref[...] = xs_ref[...] + mm_ref[...] + ob_ref[0]

    @pl.when(p > 0)
    def _():
        def ln2(i, _):
            r = pl.ds(pl.multiple_of(i * npad, 16), npad)
            h_ref[r] = _ln(xs_ref[r], ln2s_ref[0], ln2b_ref[0]).astype(bf16)
            return 0

        lax.fori_loop(0, BATCH, ln2, 0)
        hid = jnp.dot(h_ref[...], w1_ref[0],
                      preferred_element_type=f32) + b1_ref[0]
        hg_ref[...] = (hid * 0.5
                       * (1.0 + lax.erf(hid * (2.0 ** -0.5)))).astype(bf16)
        mm_ref[...] = jnp.dot(hg_ref[...], w2_ref[0],
                              preferred_element_type=f32)

        @pl.when(p == 1)
        def _():
            acc_ref[...] = mm_ref[...]

        @pl.when(p > 1)
        def _():
            acc_ref[...] = acc_ref[...] + mm_ref[...]

        @pl.when(p == NPH - 1)
        def _():
            xs_ref[...] = xs_ref[...] + acc_ref[...] + b2_ref[0]

    @pl.when((li == nlayers - 1) & (p == NPH - 1))
    def _():
        if stage1:
            def finish(i, _):
                xo = xs_ref[pl.ds(pl.multiple_of(i * npad, 16), npad)]
                out_ref[i] = xo
                a = jnp.sum(jnp.abs(xo), axis=-1, keepdims=True)
                rid = lax.broadcasted_iota(i32, (npad, 1), 0)
                valid = (rid >= 1) & (rid <= NPATCH)
                a = jnp.where(valid, a, -1.0)
                nrm_ref[pl.ds(i, 1), :] = jnp.transpose(a)
                return 0

            lax.fori_loop(0, BATCH, finish, 0)
        else:
            cls = jnp.concatenate(
                [xs_ref[s * npad:s * npad + 1, :] for s in range(BATCH)],
                axis=0)
            hcls = _ln(cls, ns_ref[...], nb_ref[...])
            out_ref[...] = _mm(hcls, hw_ref[...]) + hb_ref[...]


def _stage_call(npad, nreal, stage1, nlayers, xin, wp, extra=()):
    (ln1s, ln1b, qw, qb, ow, ob, ln2s, ln2b, w1, b1, w2, b2) = wp
    body = functools.partial(_stage_body, npad, nreal, stage1, nlayers)
    const = lambda a: pl.BlockSpec(a.shape, lambda l, p: (0,) * a.ndim)
    perl2 = lambda a: pl.BlockSpec((1,) + a.shape[1:], lambda l, p: (l,) + (0,) * (a.ndim - 1))
    c01 = lambda p: jnp.maximum(p - 1, 0)
    ins = [xin]
    in_specs = [const(xin)]
    if stage1:
        pwh, posc = extra[:2]
        ins += [pwh, posc]
        in_specs += [const(pwh), const(posc)]
    ins += [ln1s, ln1b, qw, qb, ow, ob, ln2s, ln2b]
    in_specs += [perl2(ln1s), perl2(ln1b), perl2(qw), perl2(qb),
                 perl2(ow), perl2(ob), perl2(ln2s), perl2(ln2b)]
    ins += [w1, b1, w2, b2]
    in_specs += [
        pl.BlockSpec((1, D, HCHUNK), lambda l, p: (l, 0, c01(p))),
        pl.BlockSpec((1, 1, HCHUNK), lambda l, p: (l, 0, c01(p))),
        pl.BlockSpec((1, HCHUNK, D), lambda l, p: (l, c01(p), 0)),
        perl2(b2),
    ]
    if stage1:
        out_shape = [jax.ShapeDtypeStruct((BATCH, npad, D), f32),
                     jax.ShapeDtypeStruct((BATCH, N1), f32)]
        out_specs = [pl.BlockSpec((BATCH, npad, D), lambda l, p: (0, 0, 0)),
                     pl.BlockSpec((BATCH, N1), lambda l, p: (0, 0))]
    else:
        ns, nb, hw, hb = extra
        ins += [ns, nb, hw, hb]
        in_specs += [const(ns), const(nb), const(hw), const(hb)]
        out_shape = [jax.ShapeDtypeStruct((BATCH, 128), f32)]
        out_specs = [pl.BlockSpec((BATCH, 128), lambda l, p: (0, 0))]
    res = pl.pallas_call(
        body,
        grid=(nlayers, NPH),
        in_specs=in_specs,
        out_specs=out_specs,
        out_shape=out_shape,
        scratch_shapes=[pltpu.VMEM((BATCH * npad, D), f32),
                        pltpu.VMEM((BATCH * npad, D), bf16),
                        pltpu.VMEM((BATCH * npad, 3 * D), bf16),
                        pltpu.VMEM((BATCH * npad, HCHUNK), bf16),
                        pltpu.VMEM((BATCH * npad, D), f32),
                        pltpu.VMEM((BATCH * npad, D), f32)],
        compiler_params=pltpu.CompilerParams(
            dimension_semantics=("arbitrary", "arbitrary")),
    )(*ins)
    return res[0] if len(res) == 1 else tuple(res)


_GDN = lax.GatherDimensionNumbers(
    offset_dims=(), collapsed_slice_dims=(0,), start_index_map=(0,))


def _shift_up(c, sh, lanes):
    idx = jnp.maximum(lanes - sh, 0)
    g = lax.gather(c, idx[:, None], _GDN, (1,),
                   mode=lax.GatherScatterMode.PROMISE_IN_BOUNDS)
    return jnp.where(lanes >= sh, g, 0)


def _prefix(m, lanes):
    c = m.astype(i32)
    for sh in (1, 2, 4, 8):
        c = c + _shift_up(c, sh, lanes)
    return c


def _sc_prune(keys, xflat):
    mesh = plsc.VectorSubcoreMesh(core_axis_name="c", subcore_axis_name="s")
    nchunks = N1 // 16

    @functools.partial(
        pl.kernel,
        out_type=jax.ShapeDtypeStruct((BATCH, N2, D), f32),
        mesh=mesh,
        scratch_types=[
            pltpu.VMEM((N1,), i32),
            pltpu.VMEM((N2,), i32),
            pltpu.VMEM((N2, D), f32),
            pltpu.SemaphoreType.DMA,
        ],
        compiler_params=pltpu.CompilerParams(needs_layout_passes=False),
    )
    def k(keys_hbm, x_hbm, out_hbm, keys_v, idx_v, rows_v, sem):
        wid = lax.axis_index("s") * 2 + lax.axis_index("c")

        @pl.when(wid < BATCH)
        def _():
            b = wid
            pltpu.sync_copy(keys_hbm.at[b], keys_v)
            lanes = lax.iota(i32, 16)
            nk = jnp.full((16,), N_KEEP, i32)

            def count_ge(t):
                acc = jnp.zeros((16,), i32)
                for kk in range(nchunks):
                    key = keys_v[pl.ds(kk * 16, 16)]
                    acc = acc + plsc.all_reduce_population_count(key >= t)
                return acc

            def bs_step(_, carry):
                lo, hi = carry
                mid = lo + lax.shift_right_arithmetic(
                    hi - lo + 1, jnp.full((16,), 1, i32))
                big = count_ge(mid) >= nk
                return (jnp.where(big, mid, lo), jnp.where(big, hi, mid - 1))

            lo, hi = lax.fori_loop(
                0, 31, bs_step,
                (jnp.zeros((16,), i32), jnp.full((16,), 0x7F800000, i32)))
            vstar = lo
            need = nk - count_ge(vstar + 1)

            clsrow = jnp.full((16,), b * N1, i32)
            for kk in range(N2 // 16):
                idx_v[pl.ds(kk * 16, 16)] = clsrow

            running = jnp.zeros((16,), i32)
            eq_seen = jnp.zeros((16,), i32)
            for kk in range(nchunks):
                key = keys_v[pl.ds(kk * 16, 16)]
                m_gt = key > vstar
                m_eq = key == vstar
                eq_rank = _prefix(m_eq, lanes) + eq_seen
                keep = m_gt | (m_eq & (eq_rank <= need))
                slot = _prefix(keep, lanes) + running
                ids = b * N1 + kk * 16 + lanes
                plsc.store_scatter(idx_v, [slot], ids, mask=keep)
                running = running + plsc.all_reduce_population_count(keep)
                eq_seen = eq_seen + plsc.all_reduce_population_count(m_eq)

            pltpu.async_copy(x_hbm.at[idx_v], rows_v, sem).wait()
            pltpu.sync_copy(rows_v, out_hbm.at[b])

    return k(keys, xflat)


def kernel(image, patch_W, patch_b, cls_token, pos_embed, ln1_s, ln1_b, qkv_W,
           qkv_b, proj_W, proj_b, ln2_s, ln2_b, mlp_W1, mlp_b1, mlp_W2,
           mlp_b2, norm_s, norm_b, head_W, head_b):
    xu = image.reshape(BATCH, 3, G, P, G, P).transpose(0, 2, 4, 1, 3, 5)
    xu = xu.reshape(BATCH, NPATCH, 3 * P * P)
    xu_pad = jnp.pad(xu, ((0, 0), (1, N1 - NTOK), (0, 0))).astype(bf16)

    posc = jnp.zeros((N1, D), f32)
    posc = posc.at[0].set(cls_token[0, 0] + pos_embed[0, 0])
    posc = posc.at[1:NTOK].set(pos_embed[0, 1:] + patch_b)

    pwh = patch_W.astype(bf16)
    qwh = qkv_W.astype(bf16)
    owh = proj_W.astype(bf16)
    w2h = mlp_W2.astype(bf16)
    hWp = jnp.zeros((D, 128), f32).at[:, :head_W.shape[1]].set(head_W)
    hwh = hWp.astype(bf16)
    hbp = jnp.zeros((1, 128), f32).at[0, :head_b.shape[0]].set(head_b)

    r2 = lambda a: a.reshape(1, -1)

    def wp2(lo, hi):
        sl = slice(lo, hi)
        return (ln1_s[sl, None], ln1_b[sl, None], qwh[sl], qkv_b[sl, None],
                owh[sl], proj_b[sl, None], ln2_s[sl, None], ln2_b[sl, None],
                mlp_W1[sl].astype(bf16), mlp_b1[sl, None], w2h[sl],
                mlp_b2[sl, None])

    x, norms = _stage_call(N1, NTOK, True, PRUNE_LAYER + 1, xu_pad,
                           wp2(0, PRUNE_LAYER + 1), extra=(pwh, posc))

    keys = lax.bitcast_convert_type(norms, i32)
    x2 = _sc_prune(keys, x.reshape(BATCH * N1, D))

    logits = _stage_call(N2, NREAL2, False, L - PRUNE_LAYER - 1, x2,
                         wp2(PRUNE_LAYER + 1, L),
                         extra=(r2(norm_s), r2(norm_b), hwh, hbp))
    return logits[:, :head_W.shape[1]]
